# baseline (speedup 1.0000x reference)
import jax
import jax.numpy as jnp
from jax import lax
from jax.experimental import pallas as pl
from jax.experimental.pallas import tpu as pltpu

M = 65536
UNITS = 256
IN_DIM = 256
B = 32
CM = 2048
NCH = M // CM
CMA = 1024
NCHA = M // CMA
NSTEPS = NCHA + NCH + 1


def _body(inp_ref, read_ref, h_ref, c_ref, k_ref, rk_ref, bias_ref, wg_ref,
          uw_ref, rwa_ref, lua_ref, mem_ref,
          nr_ref, hout_ref, cout_ref, lt_ref,
          us_ref, ws_ref, runmin_ref, runidx_ref, minv_ref, key_ref,
          nkey_ref, rowsm_ref):
    i = pl.program_id(0)

    @pl.when(i < NCHA)
    def _():
        j = i

        @pl.when(j == 0)
        def _():
            runmin_ref[...] = jnp.full((1, B), jnp.inf, jnp.float32)
            runidx_ref[...] = jnp.full((1, B), -1, jnp.int32)

        wg = wg_ref[...]
        uw = uw_ref[...]
        rw = rwa_ref[...]
        lu = lua_ref[...]
        ww = wg * rw + (1.0 - wg) * lu
        usage = 0.95 * uw + rw + ww
        us_ref[pl.ds(j * CMA, CMA), :] = usage
        ws_ref[pl.ds(j * CMA, CMA), :] = ww.astype(jnp.bfloat16)

        colmin = jnp.min(usage, axis=0, keepdims=True)
        gid = j * CMA + lax.broadcasted_iota(jnp.int32, (CMA, B), 0)
        idxc = jnp.max(jnp.where(usage == colmin, gid, -1),
                       axis=0, keepdims=True)
        rm = runmin_ref[...]
        ri = runidx_ref[...]
        runmin_ref[...] = jnp.minimum(colmin, rm)
        runidx_ref[...] = jnp.where(
            colmin < rm, idxc,
            jnp.where(colmin == rm, jnp.maximum(idxc, ri), ri))

    @pl.when(i == NCHA)
    def _():
        rm = runmin_ref[...]
        ri = runidx_ref[...]
        m = jnp.min(rm)
        lane = lax.broadcasted_iota(jnp.int32, (1, B), 1)
        i_nth = jnp.min(jnp.where(rm == m, lane, B))
        rowsm_ref[0] = jnp.sum(jnp.where(lane == i_nth, ri, 0))
        minv_ref[...] = rm

        x = inp_ref[...]
        rd = read_ref[...]
        z = jnp.dot(x, k_ref[:IN_DIM, :], preferred_element_type=jnp.float32)
        z = z + jnp.dot(rd, k_ref[IN_DIM:, :],
                        preferred_element_type=jnp.float32)
        z = z + jnp.dot(h_ref[...], rk_ref[...],
                        preferred_element_type=jnp.float32)
        z = z + bias_ref[...]
        zi = z[:, :UNITS]
        zf = z[:, UNITS:2 * UNITS]
        zc = z[:, 2 * UNITS:3 * UNITS]
        zo = z[:, 3 * UNITS:]
        i_g = jax.nn.sigmoid(zi)
        f_g = jax.nn.sigmoid(zf)
        o_g = jax.nn.sigmoid(zo)
        c_new = f_g * c_ref[...] + i_g * jnp.tanh(zc)
        h_new = o_g * jnp.tanh(c_new)
        cout_ref[...] = c_new
        hout_ref[...] = h_new
        key_ref[...] = h_new.astype(jnp.bfloat16)
        nkey = h_new / jnp.sqrt(
            jnp.maximum(jnp.sum(h_new * h_new, axis=1, keepdims=True), 1e-12))
        nkey_ref[...] = nkey.astype(jnp.bfloat16)

    @pl.when(i > NCHA)
    def _():
        j = i - NCHA - 1
        ww = ws_ref[pl.ds(j * CM, CM), :]
        usage = us_ref[pl.ds(j * CM, CM), :]
        lt_ref[...] = (usage <= minv_ref[...]).astype(jnp.float32)

        row = rowsm_ref[0]
        gid = j * CM + lax.broadcasted_iota(jnp.int32, (CM, 1), 0)
        memb = jnp.where(gid == row, 0.0, float(B) * mem_ref[...])
        memb = memb + jnp.dot(ww, key_ref[...],
                              preferred_element_type=jnp.float32)
        membb = memb.astype(jnp.bfloat16)
        inv = 1.0 / jnp.sqrt(
            jnp.maximum(jnp.sum(memb * memb, axis=1, keepdims=True), 1e-12))
        cos = lax.dot_general(
            membb, nkey_ref[...], (((1,), (1,)), ((), ())),
            preferred_element_type=jnp.float32) * inv
        e = jnp.exp(cos)
        w = e / jnp.sum(e, axis=1, keepdims=True)
        contrib = lax.dot_general(
            w.astype(jnp.bfloat16), membb, (((0,), (0,)), ((), ())),
            preferred_element_type=jnp.float32)

        @pl.when(j == 0)
        def _():
            nr_ref[...] = contrib

        @pl.when(j > 0)
        def _():
            nr_ref[...] = nr_ref[...] + contrib


def _run(inputs, read, h, c, kern, rkern, bias2, wg, memory, uw, rw, lu):
    const = lambda i: (0, 0)
    blka = lambda i: (jnp.minimum(i, NCHA - 1), 0)
    blkb = lambda i: (jnp.clip(i - NCHA - 1, 0, NCH - 1), 0)
    return pl.pallas_call(
        _body,
        grid=(NSTEPS,),
        in_specs=[
            pl.BlockSpec((B, IN_DIM), const),
            pl.BlockSpec((B, UNITS), const),
            pl.BlockSpec((B, UNITS), const),
            pl.BlockSpec((B, UNITS), const),
            pl.BlockSpec((IN_DIM + UNITS, 4 * UNITS), const),
            pl.BlockSpec((UNITS, 4 * UNITS), const),
            pl.BlockSpec((1, 4 * UNITS), const),
            pl.BlockSpec((1, B), const),
            pl.BlockSpec((CMA, B), blka),
            pl.BlockSpec((CMA, B), blka),
            pl.BlockSpec((CMA, B), blka),
            pl.BlockSpec((CM, UNITS), blkb),
        ],
        out_specs=[
            pl.BlockSpec((B, UNITS), const),
            pl.BlockSpec((B, UNITS), const),
            pl.BlockSpec((B, UNITS), const),
            pl.BlockSpec((CM, B), blkb),
        ],
        out_shape=[
            jax.ShapeDtypeStruct((B, UNITS), jnp.float32),
            jax.ShapeDtypeStruct((B, UNITS), jnp.float32),
            jax.ShapeDtypeStruct((B, UNITS), jnp.float32),
            jax.ShapeDtypeStruct((M, B), jnp.float32),
        ],
        scratch_shapes=[
            pltpu.VMEM((M, B), jnp.float32),
            pltpu.VMEM((M, B), jnp.bfloat16),
            pltpu.VMEM((1, B), jnp.float32),
            pltpu.VMEM((1, B), jnp.int32),
            pltpu.VMEM((1, B), jnp.float32),
            pltpu.VMEM((B, UNITS), jnp.bfloat16),
            pltpu.VMEM((B, UNITS), jnp.bfloat16),
            pltpu.SMEM((1,), jnp.int32),
        ],
        compiler_params=pltpu.CompilerParams(
            dimension_semantics=("arbitrary",),
            vmem_limit_bytes=100 * 1024 * 1024),
    )(inputs, read, h, c, kern, rkern, bias2, wg, uw, rw, lu, memory)


def kernel(inputs, h, c, kernel, recurrent_kernel, bias, write_gate, memory,
           read, least_used_weights, usage_weights, read_weights):
    wg = jax.nn.sigmoid(write_gate).reshape(1, B)
    bias2 = bias.reshape(1, 4 * UNITS)
    new_read, h_new, c_new, lt = _run(
        inputs, read, h, c, kernel, recurrent_kernel, bias2, wg, memory,
        usage_weights, read_weights, least_used_weights)
    return (new_read, h_new, c_new, lt)


# speedup vs baseline: 1.0555x; 1.0555x over previous
import jax
import jax.numpy as jnp
from jax import lax
from jax.experimental import pallas as pl
from jax.experimental.pallas import tpu as pltpu

M = 65536
UNITS = 256
IN_DIM = 256
B = 32
CM = 2048
NCH = M // CM
CMA = 4096
NCHA = M // CMA
NSTEPS = NCHA + NCH + 1


def _body(inp_ref, read_ref, h_ref, c_ref, k_ref, rk_ref, bias_ref, wg_ref,
          uw_ref, rwa_ref, lua_ref, mem_ref, rwb_ref, lub_ref,
          nr_ref, hout_ref, cout_ref, lt_ref,
          us_ref, runmin_ref, runidx_ref, minv_ref, key_ref, nkey_ref,
          rowsm_ref):
    i = pl.program_id(0)

    @pl.when(i < NCHA)
    def _():
        j = i

        @pl.when(j == 0)
        def _():
            runmin_ref[...] = jnp.full((1, B), jnp.inf, jnp.float32)
            runidx_ref[...] = jnp.full((1, B), -1, jnp.int32)

        wg = wg_ref[...]
        uw = uw_ref[...]
        rw = rwa_ref[...]
        lu = lua_ref[...]
        ww = wg * rw + (1.0 - wg) * lu
        usage = 0.95 * uw + rw + ww
        us_ref[pl.ds(j * CMA, CMA), :] = usage

        colmin = jnp.min(usage, axis=0, keepdims=True)
        gid = j * CMA + lax.broadcasted_iota(jnp.int32, (CMA, B), 0)
        idxc = jnp.max(jnp.where(usage == colmin, gid, -1),
                       axis=0, keepdims=True)
        rm = runmin_ref[...]
        ri = runidx_ref[...]
        runmin_ref[...] = jnp.minimum(colmin, rm)
        runidx_ref[...] = jnp.where(
            colmin < rm, idxc,
            jnp.where(colmin == rm, jnp.maximum(idxc, ri), ri))

    @pl.when(i == NCHA)
    def _():
        rm = runmin_ref[...]
        ri = runidx_ref[...]
        m = jnp.min(rm)
        lane = lax.broadcasted_iota(jnp.int32, (1, B), 1)
        i_nth = jnp.min(jnp.where(rm == m, lane, B))
        rowsm_ref[0] = jnp.sum(jnp.where(lane == i_nth, ri, 0))
        minv_ref[...] = rm

        x = inp_ref[...]
        rd = read_ref[...]
        z = jnp.dot(x, k_ref[:IN_DIM, :], preferred_element_type=jnp.float32)
        z = z + jnp.dot(rd, k_ref[IN_DIM:, :],
                        preferred_element_type=jnp.float32)
        z = z + jnp.dot(h_ref[...], rk_ref[...],
                        preferred_element_type=jnp.float32)
        z = z + bias_ref[...]
        zi = z[:, :UNITS]
        zf = z[:, UNITS:2 * UNITS]
        zc = z[:, 2 * UNITS:3 * UNITS]
        zo = z[:, 3 * UNITS:]
        i_g = jax.nn.sigmoid(zi)
        f_g = jax.nn.sigmoid(zf)
        o_g = jax.nn.sigmoid(zo)
        c_new = f_g * c_ref[...] + i_g * jnp.tanh(zc)
        h_new = o_g * jnp.tanh(c_new)
        cout_ref[...] = c_new
        hout_ref[...] = h_new
        key_ref[...] = h_new
        nkey = h_new / jnp.sqrt(
            jnp.maximum(jnp.sum(h_new * h_new, axis=1, keepdims=True), 1e-12))
        nkey_ref[...] = nkey.astype(jnp.bfloat16)

    @pl.when(i > NCHA)
    def _():
        j = i - NCHA - 1
        wg = wg_ref[...]
        rw = rwb_ref[...]
        lu = lub_ref[...]
        ww = wg * rw + (1.0 - wg) * lu
        usage = us_ref[pl.ds(j * CM, CM), :]
        lt_ref[...] = (usage <= minv_ref[...]).astype(jnp.float32)

        row = rowsm_ref[0]
        gid = j * CM + lax.broadcasted_iota(jnp.int32, (CM, 1), 0)
        memb = jnp.where(gid == row, 0.0, float(B) * mem_ref[...])
        memb = memb + jnp.dot(ww, key_ref[...],
                              preferred_element_type=jnp.float32)
        membb = memb.astype(jnp.bfloat16)
        inv = 1.0 / jnp.sqrt(
            jnp.maximum(jnp.sum(memb * memb, axis=1, keepdims=True), 1e-12))
        cos = lax.dot_general(
            membb, nkey_ref[...], (((1,), (1,)), ((), ())),
            preferred_element_type=jnp.float32) * inv
        e = jnp.exp(cos)
        w = e / jnp.sum(e, axis=1, keepdims=True)
        contrib = lax.dot_general(
            w.astype(jnp.bfloat16), membb, (((0,), (0,)), ((), ())),
            preferred_element_type=jnp.float32)

        @pl.when(j == 0)
        def _():
            nr_ref[...] = contrib

        @pl.when(j > 0)
        def _():
            nr_ref[...] = nr_ref[...] + contrib


def _run(inputs, read, h, c, kern, rkern, bias2, wg, memory, uw, rw, lu):
    const = lambda i: (0, 0)
    blka = lambda i: (jnp.minimum(i, NCHA - 1), 0)
    blkb = lambda i: (jnp.clip(i - NCHA - 1, 0, NCH - 1), 0)
    return pl.pallas_call(
        _body,
        grid=(NSTEPS,),
        in_specs=[
            pl.BlockSpec((B, IN_DIM), const),
            pl.BlockSpec((B, UNITS), const),
            pl.BlockSpec((B, UNITS), const),
            pl.BlockSpec((B, UNITS), const),
            pl.BlockSpec((IN_DIM + UNITS, 4 * UNITS), const),
            pl.BlockSpec((UNITS, 4 * UNITS), const),
            pl.BlockSpec((1, 4 * UNITS), const),
            pl.BlockSpec((1, B), const),
            pl.BlockSpec((CMA, B), blka),
            pl.BlockSpec((CMA, B), blka),
            pl.BlockSpec((CMA, B), blka),
            pl.BlockSpec((CM, UNITS), blkb),
            pl.BlockSpec((CM, B), blkb),
            pl.BlockSpec((CM, B), blkb),
        ],
        out_specs=[
            pl.BlockSpec((B, UNITS), const),
            pl.BlockSpec((B, UNITS), const),
            pl.BlockSpec((B, UNITS), const),
            pl.BlockSpec((CM, B), blkb),
        ],
        out_shape=[
            jax.ShapeDtypeStruct((B, UNITS), jnp.float32),
            jax.ShapeDtypeStruct((B, UNITS), jnp.float32),
            jax.ShapeDtypeStruct((B, UNITS), jnp.float32),
            jax.ShapeDtypeStruct((M, B), jnp.float32),
        ],
        scratch_shapes=[
            pltpu.VMEM((M, B), jnp.float32),
            pltpu.VMEM((1, B), jnp.float32),
            pltpu.VMEM((1, B), jnp.int32),
            pltpu.VMEM((1, B), jnp.float32),
            pltpu.VMEM((B, UNITS), jnp.float32),
            pltpu.VMEM((B, UNITS), jnp.bfloat16),
            pltpu.SMEM((1,), jnp.int32),
        ],
        compiler_params=pltpu.CompilerParams(
            dimension_semantics=("arbitrary",),
            vmem_limit_bytes=100 * 1024 * 1024),
    )(inputs, read, h, c, kern, rkern, bias2, wg, uw, rw, lu, memory, rw, lu)


def kernel(inputs, h, c, kernel, recurrent_kernel, bias, write_gate, memory,
           read, least_used_weights, usage_weights, read_weights):
    wg = jax.nn.sigmoid(write_gate).reshape(1, B)
    bias2 = bias.reshape(1, 4 * UNITS)
    new_read, h_new, c_new, lt = _run(
        inputs, read, h, c, kernel, recurrent_kernel, bias2, wg, memory,
        usage_weights, read_weights, least_used_weights)
    return (new_read, h_new, c_new, lt)
